# speedup vs baseline: 1.9000x; 1.9000x over previous
import functools

import jax
import jax.numpy as jnp
from jax import lax
from jax.experimental import pallas as pl
from jax.experimental.pallas import tpu as pltpu

TOPK = 256


def _indexer_body(lastq_ref, wq_ref, bq_ref, k_ref, out_ref):
    q_down = lax.dot_general(
        lastq_ref[...], wq_ref[...], (((1,), (1,)), ((), ())),
        preferred_element_type=jnp.float32,
    ) + bq_ref[...]

    rows = []
    for b in range(2):
        qb = q_down[b * 16:(b + 1) * 16, :]
        kb = k_ref[b]
        rows.append(lax.dot_general(
            qb, kb, (((1,), (1,)), ((), ())),
            preferred_element_type=jnp.float32,
        ))
    scores = jnp.concatenate(rows, axis=0)

    fuzzy = jnp.maximum(scores, 0.0)
    bits = jnp.where(fuzzy > 0.0, lax.bitcast_convert_type(fuzzy, jnp.int32),
                     jnp.int32(0))

    def step(i, cand):
        bit = 30 - i
        t = cand | (jnp.int32(1) << bit)
        cnt = jnp.sum((bits >= t).astype(jnp.int32), axis=1, keepdims=True)
        return jnp.where(cnt >= TOPK, t, cand)

    cand0 = jnp.zeros((bits.shape[0], 1), dtype=jnp.int32)
    kth = lax.fori_loop(0, 31, step, cand0)

    out_ref[...] = jnp.where(bits >= kth, fuzzy, 0.0)


@jax.jit
def _run(last_q, Wq, bq, K):
    out = pl.pallas_call(
        _indexer_body,
        out_shape=jax.ShapeDtypeStruct((32, 2048), jnp.float32),
    )(last_q, Wq, bq, K)
    return out


def kernel(Q, K_down, V_down, Wq, bq):
    last_q = Q[:, :, -1, :].reshape(32, 1024)
    K = K_down[:, 0, :, :]
    out = _run(last_q, Wq, bq.reshape(1, 256), K)
    return out.reshape(2, 16, 2048)
